# Initial kernel scaffold; baseline (speedup 1.0000x reference)
import jax
import jax.numpy as jnp
from jax.experimental import pallas as pl


def kernel(points, features, W1, W2, g1, b1, g2, b2):
    raise NotImplementedError("write your pallas kernel here")


# speedup vs baseline: 6.7444x; 6.7444x over previous
import functools

import jax
import jax.numpy as jnp
from jax import lax
from jax.experimental import pallas as pl
from jax.experimental.pallas import tpu as pltpu
from jax.experimental.pallas import tpu_sc as plsc

_RT = 256
_GT = 1024
_NEG = -3.0e38


def _knn_uv_body(k_, n_, pts_full, pts_rows, f_blk, a_mat, w1b, idx_ref,
                 u_ref, v_ref):
    b = pl.program_id(0)
    pf = pts_full[0]
    pr = pts_rows[0]
    xx_c = jnp.sum(pf * pf, axis=0, keepdims=True)
    xx_r = jnp.sum(pr * pr, axis=0)[:, None]
    acc = lax.dot_general(pr, pf, (((0,), (0,)), ((), ())),
                          preferred_element_type=jnp.float32)
    inner = -2.0 * acc
    P = -xx_r - inner - xx_c
    iota = lax.broadcasted_iota(jnp.int32, (_RT, n_), 1)
    cols = []
    for j in range(k_ + 1):
        m = jnp.max(P, axis=1, keepdims=True)
        sel = jnp.where(P == m, iota, n_)
        jmin = jnp.min(sel, axis=1, keepdims=True)
        if j > 0:
            cols.append(jmin)
        P = jnp.where(iota == jmin, _NEG, P)
    idx_ref[0] = jnp.concatenate(cols, axis=1) + b * n_
    u_ref[0] = jnp.dot(a_mat[...], f_blk[0],
                       preferred_element_type=jnp.float32)
    v_ref[0] = jnp.dot(w1b[...], f_blk[0],
                       preferred_element_type=jnp.float32)


def _knn_uv(points, features, a_mat, w1b, k_):
    bb, _, n_ = points.shape
    c_ = w1b.shape[0]
    grid = (bb, n_ // _RT)
    return pl.pallas_call(
        functools.partial(_knn_uv_body, k_, n_),
        grid=grid,
        in_specs=[
            pl.BlockSpec((1, 3, n_), lambda b, t: (b, 0, 0)),
            pl.BlockSpec((1, 3, _RT), lambda b, t: (b, 0, t)),
            pl.BlockSpec((1, c_, _RT), lambda b, t: (b, 0, t)),
            pl.BlockSpec((c_, c_), lambda b, t: (0, 0)),
            pl.BlockSpec((c_, c_), lambda b, t: (0, 0)),
        ],
        out_specs=[
            pl.BlockSpec((1, _RT, k_), lambda b, t: (b, t, 0)),
            pl.BlockSpec((1, c_, _RT), lambda b, t: (b, 0, t)),
            pl.BlockSpec((1, c_, _RT), lambda b, t: (b, 0, t)),
        ],
        out_shape=[
            jax.ShapeDtypeStruct((bb, n_, k_), jnp.int32),
            jax.ShapeDtypeStruct((bb, c_, n_), jnp.float32),
            jax.ShapeDtypeStruct((bb, c_, n_), jnp.float32),
        ],
    )(points, points, features, a_mat, w1b)


def _sc_gather(v_flat, idx_flat):
    m_, c_ = idx_flat.shape[0], v_flat.shape[1]
    info = plsc.get_sparse_core_info()
    nw = info.num_cores * info.num_subcores
    per_w = m_ // nw
    ch = 1024
    n_ch = per_w // ch
    mesh = plsc.VectorSubcoreMesh(core_axis_name="c", subcore_axis_name="s")

    @functools.partial(
        pl.kernel, mesh=mesh,
        compiler_params=pltpu.CompilerParams(use_tc_tiling_on_sc=False),
        out_type=jax.ShapeDtypeStruct((m_, c_), jnp.float32),
        scratch_types=[
            pltpu.VMEM((ch,), jnp.int32),
            pltpu.VMEM((ch, c_), jnp.float32),
            pltpu.SemaphoreType.DMA,
        ],
    )
    def k(v_hbm, idx_hbm, out_hbm, idx_v, rows_v, sem):
        wid = lax.axis_index("s") * info.num_cores + lax.axis_index("c")
        base = wid * per_w

        def body(i, carry):
            off = base + i * ch
            pltpu.sync_copy(idx_hbm.at[pl.ds(off, ch)], idx_v)
            pltpu.async_copy(v_hbm.at[idx_v], rows_v, sem).wait()
            pltpu.sync_copy(rows_v, out_hbm.at[pl.ds(off, ch)])
            return carry

        lax.fori_loop(0, n_ch, body, 0)

    return k(v_flat, idx_flat)


def _pass_a_body(g_ref, u_ref, s1_ref, s2_ref):
    t = pl.program_id(0)
    g = g_ref[...]
    u = u_ref[...]
    r = _GT // 4
    u4 = jnp.concatenate([u, u, u, u], axis=1)
    u4 = jnp.broadcast_to(u4[:, None, :], (r, 4, 256)).reshape(_GT, 256)
    x = g + u4
    p1 = jnp.sum(x.reshape(8, _GT // 8, 256), axis=1)
    p2 = jnp.sum((x * x).reshape(8, _GT // 8, 256), axis=1)

    @pl.when(t == 0)
    def _():
        s1_ref[...] = jnp.zeros_like(s1_ref)
        s2_ref[...] = jnp.zeros_like(s2_ref)

    s1_ref[...] += p1
    s2_ref[...] += p2


def _pass_a(g4, u_flat):
    m4 = g4.shape[0]
    grid = (m4 // _GT,)
    return pl.pallas_call(
        _pass_a_body,
        grid=grid,
        in_specs=[
            pl.BlockSpec((_GT, 256), lambda t: (t, 0)),
            pl.BlockSpec((_GT // 4, 64), lambda t: (t, 0)),
        ],
        out_specs=[
            pl.BlockSpec((8, 256), lambda t: (0, 0)),
            pl.BlockSpec((8, 256), lambda t: (0, 0)),
        ],
        out_shape=[
            jax.ShapeDtypeStruct((8, 256), jnp.float32),
            jax.ShapeDtypeStruct((8, 256), jnp.float32),
        ],
    )(g4, u_flat)


def _pass_b_body(g_ref, u_ref, ab_ref, sy_ref, syy_ref):
    t = pl.program_id(0)
    g = g_ref[...]
    u = u_ref[...]
    r = _GT // 4
    u4 = jnp.concatenate([u, u, u, u], axis=1)
    u4 = jnp.broadcast_to(u4[:, None, :], (r, 4, 256)).reshape(_GT, 256)
    a1 = ab_ref[0:1, :]
    c1 = ab_ref[1:2, :]
    y = jnp.maximum(a1 * (g + u4) + c1, 0.0)
    py = jnp.sum(y.reshape(8, _GT // 8, 256), axis=1)
    pyy = lax.dot_general(y, y, (((0,), (0,)), ((), ())),
                          preferred_element_type=jnp.float32)

    @pl.when(t == 0)
    def _():
        sy_ref[...] = jnp.zeros_like(sy_ref)
        syy_ref[...] = jnp.zeros_like(syy_ref)

    sy_ref[...] += py
    syy_ref[...] += pyy


def _pass_b(g4, u_flat, ab):
    m4 = g4.shape[0]
    grid = (m4 // _GT,)
    return pl.pallas_call(
        _pass_b_body,
        grid=grid,
        in_specs=[
            pl.BlockSpec((_GT, 256), lambda t: (t, 0)),
            pl.BlockSpec((_GT // 4, 64), lambda t: (t, 0)),
            pl.BlockSpec((2, 256), lambda t: (0, 0)),
        ],
        out_specs=[
            pl.BlockSpec((8, 256), lambda t: (0, 0)),
            pl.BlockSpec((256, 256), lambda t: (0, 0)),
        ],
        out_shape=[
            jax.ShapeDtypeStruct((8, 256), jnp.float32),
            jax.ShapeDtypeStruct((256, 256), jnp.float32),
        ],
    )(g4, u_flat, ab)


def _pass_c_body(g_ref, u_ref, ab_ref, w2_ref, f_ref, out_ref):
    g = g_ref[...]
    u = u_ref[...]
    r = _GT // 4
    u4 = jnp.concatenate([u, u, u, u], axis=1)
    u4 = jnp.broadcast_to(u4[:, None, :], (r, 4, 256)).reshape(_GT, 256)
    a1 = ab_ref[0:1, :]
    c1 = ab_ref[1:2, :]
    c2 = ab_ref[2:3, :]
    y = jnp.maximum(a1 * (g + u4) + c1, 0.0)
    z = jnp.dot(y, w2_ref[...],
                preferred_element_type=jnp.float32) + c2
    w = jnp.maximum(z, 0.0)
    t4 = jnp.sum(w.reshape(r, 4, 256), axis=1)
    s = t4[:, 0:64] + t4[:, 64:128] + t4[:, 128:192] + t4[:, 192:256]
    out_ref[...] = jnp.maximum(f_ref[...] + s * (1.0 / 16.0), 0.0)


def _pass_c(g4, u_flat, ab, w2blk, f_t):
    m4 = g4.shape[0]
    grid = (m4 // _GT,)
    return pl.pallas_call(
        _pass_c_body,
        grid=grid,
        in_specs=[
            pl.BlockSpec((_GT, 256), lambda t: (t, 0)),
            pl.BlockSpec((_GT // 4, 64), lambda t: (t, 0)),
            pl.BlockSpec((4, 256), lambda t: (0, 0)),
            pl.BlockSpec((256, 256), lambda t: (0, 0)),
            pl.BlockSpec((_GT // 4, 64), lambda t: (t, 0)),
        ],
        out_specs=pl.BlockSpec((_GT // 4, 64), lambda t: (t, 0)),
        out_shape=jax.ShapeDtypeStruct((m4 // 4, 64), jnp.float32),
    )(g4, u_flat, ab, w2blk, f_t)


def kernel(points, features, W1, W2, g1, b1, g2, b2):
    eps = jnp.float32(1e-5)
    bb, _, n_ = points.shape
    c_ = features.shape[1]
    k_ = 16
    m_ = bb * n_ * k_

    w1a, w1b = W1[:, :c_], W1[:, c_:]
    a_mat = w1a - w1b

    idx, u_bcn, v_bcn = _knn_uv(points, features, a_mat, w1b, k_)
    u_flat = u_bcn.transpose(0, 2, 1).reshape(bb * n_, c_)
    v_flat = v_bcn.transpose(0, 2, 1).reshape(bb * n_, c_)
    idx_flat = idx.reshape(m_)

    g_rows = _sc_gather(v_flat, idx_flat)
    g4 = g_rows.reshape(m_ // 4, 4 * c_)

    s1r, s2r = _pass_a(g4, u_flat)
    sum1 = s1r.sum(axis=0).reshape(4, c_).sum(axis=0)
    sumsq1 = s2r.sum(axis=0).reshape(4, c_).sum(axis=0)
    mf = jnp.float32(m_)
    m1 = sum1 / mf
    var1 = sumsq1 / mf - m1 * m1
    a1 = g1 / jnp.sqrt(var1 + eps)
    c1 = b1 - a1 * m1
    ab1 = jnp.stack([jnp.tile(a1, 4), jnp.tile(c1, 4)])

    syr, syy = _pass_b(g4, u_flat, ab1)
    sy = syr.sum(axis=0).reshape(4, c_).sum(axis=0)
    syy64 = (syy[0:64, 0:64] + syy[64:128, 64:128]
             + syy[128:192, 128:192] + syy[192:256, 192:256])
    m2 = (W2 @ sy) / mf
    e2 = jnp.einsum('oc,cd,od->o', W2, syy64, W2) / mf
    var2 = e2 - m2 * m2
    a2 = g2 / jnp.sqrt(var2 + eps)
    c2 = b2 - a2 * m2
    w2p = a2[:, None] * W2
    zero = jnp.zeros((c_, c_), jnp.float32)
    w2t = w2p.T
    w2blk = jnp.block([
        [w2t, zero, zero, zero],
        [zero, w2t, zero, zero],
        [zero, zero, w2t, zero],
        [zero, zero, zero, w2t],
    ])
    ab2 = jnp.concatenate(
        [ab1, jnp.tile(c2, 4)[None, :], jnp.zeros((1, 4 * c_), jnp.float32)])

    f_t = features.transpose(0, 2, 1).reshape(bb * n_, c_)
    out_t = _pass_c(g4, u_flat, ab2, w2blk, f_t)
    return out_t.reshape(bb, n_, c_).transpose(0, 2, 1)


# speedup vs baseline: 7.4423x; 1.1035x over previous
import functools

import jax
import jax.numpy as jnp
from jax import lax
from jax.experimental import pallas as pl
from jax.experimental.pallas import tpu as pltpu
from jax.experimental.pallas import tpu_sc as plsc

_RT = 256
_GT = 1024
_NEG = -3.0e38


def _knn_uv_body(k_, n_, pts_full, pts_rows, f_blk, a_mat, w1b, idx_ref,
                 u_ref, v_ref):
    b = pl.program_id(0)
    pf = pts_full[0]
    pr = pts_rows[0]
    xx_c = jnp.sum(pf * pf, axis=0, keepdims=True)
    xx_r = jnp.sum(pr * pr, axis=0)[:, None]
    acc = lax.dot_general(pr, pf, (((0,), (0,)), ((), ())),
                          preferred_element_type=jnp.float32)
    inner = -2.0 * acc
    P = -xx_r - inner - xx_c
    iota_f = lax.broadcasted_iota(jnp.int32, (_RT, n_), 1).astype(jnp.float32)
    cols = []
    for j in range(k_ + 1):
        m = jnp.max(P, axis=1, keepdims=True)
        sel = jnp.where(P == m, iota_f, 3.0e38)
        jmin = jnp.min(sel, axis=1, keepdims=True)
        if j > 0:
            cols.append(jmin)
        P = jnp.where(sel == jmin, _NEG, P)
    idxf = jnp.concatenate(cols, axis=1)
    idx_ref[0] = idxf.astype(jnp.int32) + b * n_
    u_ref[0] = jnp.dot(a_mat[...], f_blk[0],
                       preferred_element_type=jnp.float32)
    v_ref[0] = jnp.dot(w1b[...], f_blk[0],
                       preferred_element_type=jnp.float32)


def _knn_uv(points, features, a_mat, w1b, k_):
    bb, _, n_ = points.shape
    c_ = w1b.shape[0]
    grid = (bb, n_ // _RT)
    return pl.pallas_call(
        functools.partial(_knn_uv_body, k_, n_),
        grid=grid,
        in_specs=[
            pl.BlockSpec((1, 3, n_), lambda b, t: (b, 0, 0)),
            pl.BlockSpec((1, 3, _RT), lambda b, t: (b, 0, t)),
            pl.BlockSpec((1, c_, _RT), lambda b, t: (b, 0, t)),
            pl.BlockSpec((c_, c_), lambda b, t: (0, 0)),
            pl.BlockSpec((c_, c_), lambda b, t: (0, 0)),
        ],
        out_specs=[
            pl.BlockSpec((1, _RT, k_), lambda b, t: (b, t, 0)),
            pl.BlockSpec((1, c_, _RT), lambda b, t: (b, 0, t)),
            pl.BlockSpec((1, c_, _RT), lambda b, t: (b, 0, t)),
        ],
        out_shape=[
            jax.ShapeDtypeStruct((bb, n_, k_), jnp.int32),
            jax.ShapeDtypeStruct((bb, c_, n_), jnp.float32),
            jax.ShapeDtypeStruct((bb, c_, n_), jnp.float32),
        ],
    )(points, points, features, a_mat, w1b)


def _sc_gather(v_flat, idx_flat):
    m_, c_ = idx_flat.shape[0], v_flat.shape[1]
    info = plsc.get_sparse_core_info()
    nw = info.num_cores * info.num_subcores
    per_w = m_ // nw
    ch = 1024
    n_ch = per_w // ch
    mesh = plsc.VectorSubcoreMesh(core_axis_name="c", subcore_axis_name="s")

    @functools.partial(
        pl.kernel, mesh=mesh,
        compiler_params=pltpu.CompilerParams(use_tc_tiling_on_sc=False),
        out_type=jax.ShapeDtypeStruct((m_, c_), jnp.float32),
        scratch_types=[
            pltpu.VMEM((ch,), jnp.int32),
            pltpu.VMEM((ch, c_), jnp.float32),
            pltpu.SemaphoreType.DMA,
        ],
    )
    def k(v_hbm, idx_hbm, out_hbm, idx_v, rows_v, sem):
        wid = lax.axis_index("s") * info.num_cores + lax.axis_index("c")
        base = wid * per_w

        def body(i, carry):
            off = base + i * ch
            pltpu.sync_copy(idx_hbm.at[pl.ds(off, ch)], idx_v)
            pltpu.async_copy(v_hbm.at[idx_v], rows_v, sem).wait()
            pltpu.sync_copy(rows_v, out_hbm.at[pl.ds(off, ch)])
            return carry

        lax.fori_loop(0, n_ch, body, 0)

    return k(v_flat, idx_flat)


def _pass_a_body(g_ref, u_ref, s1_ref, s2_ref):
    t = pl.program_id(0)
    g = g_ref[...]
    u = u_ref[...]
    r = _GT // 4
    u4 = jnp.concatenate([u, u, u, u], axis=1)
    u4 = jnp.broadcast_to(u4[:, None, :], (r, 4, 256)).reshape(_GT, 256)
    x = g + u4
    p1 = jnp.sum(x.reshape(8, _GT // 8, 256), axis=1)
    p2 = jnp.sum((x * x).reshape(8, _GT // 8, 256), axis=1)

    @pl.when(t == 0)
    def _():
        s1_ref[...] = jnp.zeros_like(s1_ref)
        s2_ref[...] = jnp.zeros_like(s2_ref)

    s1_ref[...] += p1
    s2_ref[...] += p2


def _pass_a(g4, u_flat):
    m4 = g4.shape[0]
    grid = (m4 // _GT,)
    return pl.pallas_call(
        _pass_a_body,
        grid=grid,
        in_specs=[
            pl.BlockSpec((_GT, 256), lambda t: (t, 0)),
            pl.BlockSpec((_GT // 4, 64), lambda t: (t, 0)),
        ],
        out_specs=[
            pl.BlockSpec((8, 256), lambda t: (0, 0)),
            pl.BlockSpec((8, 256), lambda t: (0, 0)),
        ],
        out_shape=[
            jax.ShapeDtypeStruct((8, 256), jnp.float32),
            jax.ShapeDtypeStruct((8, 256), jnp.float32),
        ],
    )(g4, u_flat)


def _pass_b_body(g_ref, u_ref, ab_ref, sy_ref, syy_ref):
    t = pl.program_id(0)
    g = g_ref[...]
    u = u_ref[...]
    r = _GT // 4
    u4 = jnp.concatenate([u, u, u, u], axis=1)
    u4 = jnp.broadcast_to(u4[:, None, :], (r, 4, 256)).reshape(_GT, 256)
    a1 = ab_ref[0:1, :]
    c1 = ab_ref[1:2, :]
    y = jnp.maximum(a1 * (g + u4) + c1, 0.0)
    py = jnp.sum(y.reshape(8, _GT // 8, 256), axis=1)
    pyy = lax.dot_general(y, y, (((0,), (0,)), ((), ())),
                          preferred_element_type=jnp.float32)

    @pl.when(t == 0)
    def _():
        sy_ref[...] = jnp.zeros_like(sy_ref)
        syy_ref[...] = jnp.zeros_like(syy_ref)

    sy_ref[...] += py
    syy_ref[...] += pyy


def _pass_b(g4, u_flat, ab):
    m4 = g4.shape[0]
    grid = (m4 // _GT,)
    return pl.pallas_call(
        _pass_b_body,
        grid=grid,
        in_specs=[
            pl.BlockSpec((_GT, 256), lambda t: (t, 0)),
            pl.BlockSpec((_GT // 4, 64), lambda t: (t, 0)),
            pl.BlockSpec((2, 256), lambda t: (0, 0)),
        ],
        out_specs=[
            pl.BlockSpec((8, 256), lambda t: (0, 0)),
            pl.BlockSpec((256, 256), lambda t: (0, 0)),
        ],
        out_shape=[
            jax.ShapeDtypeStruct((8, 256), jnp.float32),
            jax.ShapeDtypeStruct((256, 256), jnp.float32),
        ],
    )(g4, u_flat, ab)


def _pass_c_body(g_ref, u_ref, ab_ref, w2_ref, f_ref, out_ref):
    g = g_ref[...]
    u = u_ref[...]
    r = _GT // 4
    u4 = jnp.concatenate([u, u, u, u], axis=1)
    u4 = jnp.broadcast_to(u4[:, None, :], (r, 4, 256)).reshape(_GT, 256)
    a1 = ab_ref[0:1, :]
    c1 = ab_ref[1:2, :]
    c2 = ab_ref[2:3, :]
    y = jnp.maximum(a1 * (g + u4) + c1, 0.0)
    z = jnp.dot(y, w2_ref[...],
                preferred_element_type=jnp.float32) + c2
    w = jnp.maximum(z, 0.0)
    t4 = jnp.sum(w.reshape(r, 4, 256), axis=1)
    s = t4[:, 0:64] + t4[:, 64:128] + t4[:, 128:192] + t4[:, 192:256]
    out_ref[...] = jnp.maximum(f_ref[...] + s * (1.0 / 16.0), 0.0)


def _pass_c(g4, u_flat, ab, w2blk, f_t):
    m4 = g4.shape[0]
    grid = (m4 // _GT,)
    return pl.pallas_call(
        _pass_c_body,
        grid=grid,
        in_specs=[
            pl.BlockSpec((_GT, 256), lambda t: (t, 0)),
            pl.BlockSpec((_GT // 4, 64), lambda t: (t, 0)),
            pl.BlockSpec((4, 256), lambda t: (0, 0)),
            pl.BlockSpec((256, 256), lambda t: (0, 0)),
            pl.BlockSpec((_GT // 4, 64), lambda t: (t, 0)),
        ],
        out_specs=pl.BlockSpec((_GT // 4, 64), lambda t: (t, 0)),
        out_shape=jax.ShapeDtypeStruct((m4 // 4, 64), jnp.float32),
    )(g4, u_flat, ab, w2blk, f_t)


def kernel(points, features, W1, W2, g1, b1, g2, b2):
    eps = jnp.float32(1e-5)
    bb, _, n_ = points.shape
    c_ = features.shape[1]
    k_ = 16
    m_ = bb * n_ * k_

    w1a, w1b = W1[:, :c_], W1[:, c_:]
    a_mat = w1a - w1b

    idx, u_bcn, v_bcn = _knn_uv(points, features, a_mat, w1b, k_)
    u_flat = u_bcn.transpose(0, 2, 1).reshape(bb * n_, c_)
    v_flat = v_bcn.transpose(0, 2, 1).reshape(bb * n_, c_)
    idx_flat = idx.reshape(m_)

    g_rows = _sc_gather(v_flat, idx_flat)
    g4 = g_rows.reshape(m_ // 4, 4 * c_)

    s1r, s2r = _pass_a(g4, u_flat)
    sum1 = s1r.sum(axis=0).reshape(4, c_).sum(axis=0)
    sumsq1 = s2r.sum(axis=0).reshape(4, c_).sum(axis=0)
    mf = jnp.float32(m_)
    m1 = sum1 / mf
    var1 = sumsq1 / mf - m1 * m1
    a1 = g1 / jnp.sqrt(var1 + eps)
    c1 = b1 - a1 * m1
    ab1 = jnp.stack([jnp.tile(a1, 4), jnp.tile(c1, 4)])

    syr, syy = _pass_b(g4, u_flat, ab1)
    sy = syr.sum(axis=0).reshape(4, c_).sum(axis=0)
    syy64 = (syy[0:64, 0:64] + syy[64:128, 64:128]
             + syy[128:192, 128:192] + syy[192:256, 192:256])
    m2 = (W2 @ sy) / mf
    e2 = jnp.einsum('oc,cd,od->o', W2, syy64, W2) / mf
    var2 = e2 - m2 * m2
    a2 = g2 / jnp.sqrt(var2 + eps)
    c2 = b2 - a2 * m2
    w2p = a2[:, None] * W2
    zero = jnp.zeros((c_, c_), jnp.float32)
    w2t = w2p.T
    w2blk = jnp.block([
        [w2t, zero, zero, zero],
        [zero, w2t, zero, zero],
        [zero, zero, w2t, zero],
        [zero, zero, zero, w2t],
    ])
    ab2 = jnp.concatenate(
        [ab1, jnp.tile(c2, 4)[None, :], jnp.zeros((1, 4 * c_), jnp.float32)])

    f_t = features.transpose(0, 2, 1).reshape(bb * n_, c_)
    out_t = _pass_c(g4, u_flat, ab2, w2blk, f_t)
    return out_t.reshape(bb, n_, c_).transpose(0, 2, 1)


# speedup vs baseline: 8.2714x; 1.1114x over previous
import functools

import jax
import jax.numpy as jnp
from jax import lax
from jax.experimental import pallas as pl
from jax.experimental.pallas import tpu as pltpu
from jax.experimental.pallas import tpu_sc as plsc

_RT = 256
_GT = 1024
_NEG = -3.0e38


def _knn_uv_body(k_, n_, pts_full, pts_rows, f_blk, a_mat, w1b, idx_ref,
                 u_ref, v_ref):
    b = pl.program_id(0)
    pf = pts_full[0]
    pr = pts_rows[0]
    xx_c = jnp.sum(pf * pf, axis=0, keepdims=True)
    xx_r = jnp.sum(pr * pr, axis=0)[:, None]
    acc = lax.dot_general(pr, pf, (((0,), (0,)), ((), ())),
                          preferred_element_type=jnp.float32)
    inner = -2.0 * acc
    P = -xx_r - inner - xx_c
    iota_f = lax.broadcasted_iota(jnp.int32, (_RT, n_), 1).astype(jnp.float32)
    cols = []
    m = jnp.max(P, axis=1, keepdims=True)
    for j in range(k_ + 1):
        jmin = jnp.min(jnp.where(P == m, iota_f, 3.0e38), axis=1,
                       keepdims=True)
        if j > 0:
            cols.append(jmin)
        P = jnp.where(iota_f == jmin, _NEG, P)
        if j < k_:
            m = jnp.max(P, axis=1, keepdims=True)
    idxf = jnp.concatenate(cols, axis=1)
    idx_ref[0] = idxf.astype(jnp.int32) + b * n_
    u_ref[0] = jnp.dot(a_mat[...], f_blk[0],
                       preferred_element_type=jnp.float32).T
    v_ref[0] = jnp.dot(w1b[...], f_blk[0],
                       preferred_element_type=jnp.float32).T


def _knn_uv(points, features, a_mat, w1b, k_):
    bb, _, n_ = points.shape
    c_ = w1b.shape[0]
    grid = (bb, n_ // _RT)
    return pl.pallas_call(
        functools.partial(_knn_uv_body, k_, n_),
        grid=grid,
        in_specs=[
            pl.BlockSpec((1, 3, n_), lambda b, t: (b, 0, 0)),
            pl.BlockSpec((1, 3, _RT), lambda b, t: (b, 0, t)),
            pl.BlockSpec((1, c_, _RT), lambda b, t: (b, 0, t)),
            pl.BlockSpec((c_, c_), lambda b, t: (0, 0)),
            pl.BlockSpec((c_, c_), lambda b, t: (0, 0)),
        ],
        out_specs=[
            pl.BlockSpec((1, _RT, k_), lambda b, t: (b, t, 0)),
            pl.BlockSpec((1, _RT, c_), lambda b, t: (b, t, 0)),
            pl.BlockSpec((1, _RT, c_), lambda b, t: (b, t, 0)),
        ],
        out_shape=[
            jax.ShapeDtypeStruct((bb, n_, k_), jnp.int32),
            jax.ShapeDtypeStruct((bb, n_, c_), jnp.float32),
            jax.ShapeDtypeStruct((bb, n_, c_), jnp.float32),
        ],
    )(points, points, features, a_mat, w1b)


def _sc_gather(v_flat, idx_flat):
    m_, c_ = idx_flat.shape[0], v_flat.shape[1]
    info = plsc.get_sparse_core_info()
    nw = info.num_cores * info.num_subcores
    per_w = m_ // nw
    ch = 1024
    n_ch = per_w // ch
    mesh = plsc.VectorSubcoreMesh(core_axis_name="c", subcore_axis_name="s")

    @functools.partial(
        pl.kernel, mesh=mesh,
        compiler_params=pltpu.CompilerParams(use_tc_tiling_on_sc=False),
        out_type=jax.ShapeDtypeStruct((m_, c_), jnp.float32),
        scratch_types=[
            pltpu.VMEM((ch,), jnp.int32),
            pltpu.VMEM((ch, c_), jnp.float32),
            pltpu.SemaphoreType.DMA,
        ],
    )
    def k(v_hbm, idx_hbm, out_hbm, idx_v, rows_v, sem):
        wid = lax.axis_index("s") * info.num_cores + lax.axis_index("c")
        base = wid * per_w

        def body(i, carry):
            off = base + i * ch
            pltpu.sync_copy(idx_hbm.at[pl.ds(off, ch)], idx_v)
            pltpu.async_copy(v_hbm.at[idx_v], rows_v, sem).wait()
            pltpu.sync_copy(rows_v, out_hbm.at[pl.ds(off, ch)])
            return carry

        lax.fori_loop(0, n_ch, body, 0)

    return k(v_flat, idx_flat)


def _pass_a_body(g_ref, u_ref, s1_ref, s2_ref):
    t = pl.program_id(0)
    g = g_ref[...]
    u = u_ref[...]
    r = _GT // 4
    u4 = jnp.concatenate([u, u, u, u], axis=1)
    u4 = jnp.broadcast_to(u4[:, None, :], (r, 4, 256)).reshape(_GT, 256)
    x = g + u4
    p1 = jnp.sum(x.reshape(8, _GT // 8, 256), axis=1)
    p2 = jnp.sum((x * x).reshape(8, _GT // 8, 256), axis=1)

    @pl.when(t == 0)
    def _():
        s1_ref[...] = jnp.zeros_like(s1_ref)
        s2_ref[...] = jnp.zeros_like(s2_ref)

    s1_ref[...] += p1
    s2_ref[...] += p2


def _pass_a(g4, u_flat):
    m4 = g4.shape[0]
    grid = (m4 // _GT,)
    return pl.pallas_call(
        _pass_a_body,
        grid=grid,
        in_specs=[
            pl.BlockSpec((_GT, 256), lambda t: (t, 0)),
            pl.BlockSpec((_GT // 4, 64), lambda t: (t, 0)),
        ],
        out_specs=[
            pl.BlockSpec((8, 256), lambda t: (0, 0)),
            pl.BlockSpec((8, 256), lambda t: (0, 0)),
        ],
        out_shape=[
            jax.ShapeDtypeStruct((8, 256), jnp.float32),
            jax.ShapeDtypeStruct((8, 256), jnp.float32),
        ],
    )(g4, u_flat)


def _pass_b_body(g_ref, u_ref, ab_ref, sy_ref, syy_ref):
    t = pl.program_id(0)
    g = g_ref[...]
    u = u_ref[...]
    r = _GT // 4
    u4 = jnp.concatenate([u, u, u, u], axis=1)
    u4 = jnp.broadcast_to(u4[:, None, :], (r, 4, 256)).reshape(_GT, 256)
    a1 = ab_ref[0:1, :]
    c1 = ab_ref[1:2, :]
    y = jnp.maximum(a1 * (g + u4) + c1, 0.0)
    py = jnp.sum(y.reshape(8, _GT // 8, 256), axis=1)
    pyy = lax.dot_general(y, y, (((0,), (0,)), ((), ())),
                          preferred_element_type=jnp.float32)

    @pl.when(t == 0)
    def _():
        sy_ref[...] = jnp.zeros_like(sy_ref)
        syy_ref[...] = jnp.zeros_like(syy_ref)

    sy_ref[...] += py
    syy_ref[...] += pyy


def _pass_b(g4, u_flat, ab):
    m4 = g4.shape[0]
    grid = (m4 // _GT,)
    return pl.pallas_call(
        _pass_b_body,
        grid=grid,
        in_specs=[
            pl.BlockSpec((_GT, 256), lambda t: (t, 0)),
            pl.BlockSpec((_GT // 4, 64), lambda t: (t, 0)),
            pl.BlockSpec((2, 256), lambda t: (0, 0)),
        ],
        out_specs=[
            pl.BlockSpec((8, 256), lambda t: (0, 0)),
            pl.BlockSpec((256, 256), lambda t: (0, 0)),
        ],
        out_shape=[
            jax.ShapeDtypeStruct((8, 256), jnp.float32),
            jax.ShapeDtypeStruct((256, 256), jnp.float32),
        ],
    )(g4, u_flat, ab)


def _pass_c_body(g_ref, u_ref, ab_ref, w2_ref, f_ref, out_ref):
    g = g_ref[...]
    u = u_ref[...]
    r = _GT // 4
    u4 = jnp.concatenate([u, u, u, u], axis=1)
    u4 = jnp.broadcast_to(u4[:, None, :], (r, 4, 256)).reshape(_GT, 256)
    a1 = ab_ref[0:1, :]
    c1 = ab_ref[1:2, :]
    c2 = ab_ref[2:3, :]
    y = jnp.maximum(a1 * (g + u4) + c1, 0.0)
    z = jnp.dot(y, w2_ref[...],
                preferred_element_type=jnp.float32) + c2
    w = jnp.maximum(z, 0.0)
    t4 = jnp.sum(w.reshape(r, 4, 256), axis=1)
    s = t4[:, 0:64] + t4[:, 64:128] + t4[:, 128:192] + t4[:, 192:256]
    out_ref[0] = jnp.maximum(f_ref[0] + s.T * (1.0 / 16.0), 0.0)


def _pass_c(g4, u_flat, ab, w2blk, features):
    m4 = g4.shape[0]
    bb, c_, n_ = features.shape
    nt = n_ // (_GT // 4)
    grid = (m4 // _GT,)
    return pl.pallas_call(
        _pass_c_body,
        grid=grid,
        in_specs=[
            pl.BlockSpec((_GT, 256), lambda t: (t, 0)),
            pl.BlockSpec((_GT // 4, 64), lambda t: (t, 0)),
            pl.BlockSpec((4, 256), lambda t: (0, 0)),
            pl.BlockSpec((256, 256), lambda t: (0, 0)),
            pl.BlockSpec((1, c_, _GT // 4), lambda t: (t // nt, 0, t % nt)),
        ],
        out_specs=pl.BlockSpec((1, c_, _GT // 4), lambda t: (t // nt, 0, t % nt)),
        out_shape=jax.ShapeDtypeStruct((bb, c_, n_), jnp.float32),
    )(g4, u_flat, ab, w2blk, features)


def kernel(points, features, W1, W2, g1, b1, g2, b2):
    eps = jnp.float32(1e-5)
    bb, _, n_ = points.shape
    c_ = features.shape[1]
    k_ = 16
    m_ = bb * n_ * k_

    w1a, w1b = W1[:, :c_], W1[:, c_:]
    a_mat = w1a - w1b

    idx, u_bnc, v_bnc = _knn_uv(points, features, a_mat, w1b, k_)
    u_flat = u_bnc.reshape(bb * n_, c_)
    v_flat = v_bnc.reshape(bb * n_, c_)
    idx_flat = idx.reshape(m_)

    g_rows = _sc_gather(v_flat, idx_flat)
    g4 = g_rows.reshape(m_ // 4, 4 * c_)

    s1r, s2r = _pass_a(g4, u_flat)
    sum1 = s1r.sum(axis=0).reshape(4, c_).sum(axis=0)
    sumsq1 = s2r.sum(axis=0).reshape(4, c_).sum(axis=0)
    mf = jnp.float32(m_)
    m1 = sum1 / mf
    var1 = sumsq1 / mf - m1 * m1
    a1 = g1 / jnp.sqrt(var1 + eps)
    c1 = b1 - a1 * m1
    ab1 = jnp.stack([jnp.tile(a1, 4), jnp.tile(c1, 4)])

    syr, syy = _pass_b(g4, u_flat, ab1)
    sy = syr.sum(axis=0).reshape(4, c_).sum(axis=0)
    syy64 = (syy[0:64, 0:64] + syy[64:128, 64:128]
             + syy[128:192, 128:192] + syy[192:256, 192:256])
    m2 = (W2 @ sy) / mf
    e2 = jnp.einsum('oc,cd,od->o', W2, syy64, W2) / mf
    var2 = e2 - m2 * m2
    a2 = g2 / jnp.sqrt(var2 + eps)
    c2 = b2 - a2 * m2
    w2p = a2[:, None] * W2
    zero = jnp.zeros((c_, c_), jnp.float32)
    w2t = w2p.T
    w2blk = jnp.block([
        [w2t, zero, zero, zero],
        [zero, w2t, zero, zero],
        [zero, zero, w2t, zero],
        [zero, zero, zero, w2t],
    ])
    ab2 = jnp.concatenate(
        [ab1, jnp.tile(c2, 4)[None, :], jnp.zeros((1, 4 * c_), jnp.float32)])

    return _pass_c(g4, u_flat, ab2, w2blk, features)


# speedup vs baseline: 8.3163x; 1.0054x over previous
import functools

import jax
import jax.numpy as jnp
from jax import lax
from jax.experimental import pallas as pl
from jax.experimental.pallas import tpu as pltpu
from jax.experimental.pallas import tpu_sc as plsc

_RT = 256
_GT = 1024
_NEG = -3.0e38


def _knn_uv_body(k_, n_, pts_full, pts_rows, f_blk, a_mat, w1b, idx_ref,
                 u_ref, v_ref):
    b = pl.program_id(0)
    pf = pts_full[0]
    pr = pts_rows[0]
    xx_c = jnp.sum(pf * pf, axis=0, keepdims=True)
    xx_r = jnp.sum(pr * pr, axis=0)[:, None]
    acc = lax.dot_general(pr, pf, (((0,), (0,)), ((), ())),
                          preferred_element_type=jnp.float32)
    inner = -2.0 * acc
    P = -xx_r - inner - xx_c
    iota_f = lax.broadcasted_iota(jnp.int32, (_RT, n_), 1).astype(jnp.float32)
    cols = []
    m = jnp.max(P, axis=1, keepdims=True)
    for j in range(k_ + 1):
        jmin = jnp.min(jnp.where(P == m, iota_f, 3.0e38), axis=1,
                       keepdims=True)
        if j > 0:
            cols.append(jmin)
        P = jnp.where(iota_f == jmin, _NEG, P)
        if j < k_:
            m = jnp.max(P, axis=1, keepdims=True)
    idxf = jnp.concatenate(cols, axis=1)
    idx_ref[0] = idxf.astype(jnp.int32) + b * n_
    u_ref[0] = jnp.dot(a_mat[...], f_blk[0],
                       preferred_element_type=jnp.float32).T
    v_ref[0] = jnp.dot(w1b[...], f_blk[0],
                       preferred_element_type=jnp.float32).T


def _knn_uv(points, features, a_mat, w1b, k_):
    bb, _, n_ = points.shape
    c_ = w1b.shape[0]
    grid = (bb, n_ // _RT)
    return pl.pallas_call(
        functools.partial(_knn_uv_body, k_, n_),
        grid=grid,
        in_specs=[
            pl.BlockSpec((1, 3, n_), lambda b, t: (b, 0, 0)),
            pl.BlockSpec((1, 3, _RT), lambda b, t: (b, 0, t)),
            pl.BlockSpec((1, c_, _RT), lambda b, t: (b, 0, t)),
            pl.BlockSpec((c_, c_), lambda b, t: (0, 0)),
            pl.BlockSpec((c_, c_), lambda b, t: (0, 0)),
        ],
        out_specs=[
            pl.BlockSpec((1, _RT, k_), lambda b, t: (b, t, 0)),
            pl.BlockSpec((1, _RT, c_), lambda b, t: (b, t, 0)),
            pl.BlockSpec((1, _RT, c_), lambda b, t: (b, t, 0)),
        ],
        out_shape=[
            jax.ShapeDtypeStruct((bb, n_, k_), jnp.int32),
            jax.ShapeDtypeStruct((bb, n_, c_), jnp.float32),
            jax.ShapeDtypeStruct((bb, n_, c_), jnp.float32),
        ],
    )(points, points, features, a_mat, w1b)


def _sc_gather(v_flat, idx_flat):
    m_, c_ = idx_flat.shape[0], v_flat.shape[1]
    info = plsc.get_sparse_core_info()
    nw = info.num_cores * info.num_subcores
    per_w = m_ // nw
    ch = 512
    n_ch = per_w // ch
    idx3 = idx_flat.reshape(nw, n_ch, ch)
    mesh = plsc.VectorSubcoreMesh(core_axis_name="c", subcore_axis_name="s")

    @functools.partial(
        pl.kernel, mesh=mesh,
        compiler_params=pltpu.CompilerParams(use_tc_tiling_on_sc=False),
        out_type=jax.ShapeDtypeStruct((m_, c_), jnp.float32),
        scratch_types=[
            pltpu.VMEM((n_ch, ch), jnp.int32),
            pltpu.VMEM((ch, c_), jnp.float32),
            pltpu.VMEM((ch, c_), jnp.float32),
            pltpu.SemaphoreType.DMA,
            pltpu.SemaphoreType.DMA,
        ],
    )
    def k(v_hbm, idx_hbm, out_hbm, idx_v, rows0, rows1, sem0, sem1):
        wid = lax.axis_index("s") * info.num_cores + lax.axis_index("c")
        base = wid * per_w
        pltpu.sync_copy(idx_hbm.at[wid], idx_v)
        bufs = (rows0, rows1)
        sems = (sem0, sem1)
        handles = [None, None]
        handles[0] = pltpu.async_copy(v_hbm.at[idx_v.at[0]], rows0, sem0)
        for i in range(n_ch):
            if i + 1 < n_ch:
                handles[(i + 1) % 2] = pltpu.async_copy(
                    v_hbm.at[idx_v.at[i + 1]], bufs[(i + 1) % 2],
                    sems[(i + 1) % 2])
            handles[i % 2].wait()
            pltpu.sync_copy(bufs[i % 2], out_hbm.at[pl.ds(base + i * ch, ch)])

    return k(v_flat, idx3)


def _pass_a_body(g_ref, u_ref, s1_ref, s2_ref):
    t = pl.program_id(0)
    g = g_ref[...]
    u = u_ref[...]
    r = _GT // 4
    u4 = jnp.concatenate([u, u, u, u], axis=1)
    u4 = jnp.broadcast_to(u4[:, None, :], (r, 4, 256)).reshape(_GT, 256)
    x = g + u4
    p1 = jnp.sum(x.reshape(8, _GT // 8, 256), axis=1)
    p2 = jnp.sum((x * x).reshape(8, _GT // 8, 256), axis=1)

    @pl.when(t == 0)
    def _():
        s1_ref[...] = jnp.zeros_like(s1_ref)
        s2_ref[...] = jnp.zeros_like(s2_ref)

    s1_ref[...] += p1
    s2_ref[...] += p2


def _pass_a(g4, u_flat):
    m4 = g4.shape[0]
    grid = (m4 // _GT,)
    return pl.pallas_call(
        _pass_a_body,
        grid=grid,
        in_specs=[
            pl.BlockSpec((_GT, 256), lambda t: (t, 0)),
            pl.BlockSpec((_GT // 4, 64), lambda t: (t, 0)),
        ],
        out_specs=[
            pl.BlockSpec((8, 256), lambda t: (0, 0)),
            pl.BlockSpec((8, 256), lambda t: (0, 0)),
        ],
        out_shape=[
            jax.ShapeDtypeStruct((8, 256), jnp.float32),
            jax.ShapeDtypeStruct((8, 256), jnp.float32),
        ],
    )(g4, u_flat)


def _pass_b_body(g_ref, u_ref, ab_ref, sy_ref, syy_ref):
    t = pl.program_id(0)
    g = g_ref[...]
    u = u_ref[...]
    r = _GT // 4
    u4 = jnp.concatenate([u, u, u, u], axis=1)
    u4 = jnp.broadcast_to(u4[:, None, :], (r, 4, 256)).reshape(_GT, 256)
    a1 = ab_ref[0:1, :]
    c1 = ab_ref[1:2, :]
    y = jnp.maximum(a1 * (g + u4) + c1, 0.0)
    py = jnp.sum(y.reshape(8, _GT // 8, 256), axis=1)
    pyy = lax.dot_general(y, y, (((0,), (0,)), ((), ())),
                          preferred_element_type=jnp.float32)

    @pl.when(t == 0)
    def _():
        sy_ref[...] = jnp.zeros_like(sy_ref)
        syy_ref[...] = jnp.zeros_like(syy_ref)

    sy_ref[...] += py
    syy_ref[...] += pyy


def _pass_b(g4, u_flat, ab):
    m4 = g4.shape[0]
    grid = (m4 // _GT,)
    return pl.pallas_call(
        _pass_b_body,
        grid=grid,
        in_specs=[
            pl.BlockSpec((_GT, 256), lambda t: (t, 0)),
            pl.BlockSpec((_GT // 4, 64), lambda t: (t, 0)),
            pl.BlockSpec((2, 256), lambda t: (0, 0)),
        ],
        out_specs=[
            pl.BlockSpec((8, 256), lambda t: (0, 0)),
            pl.BlockSpec((256, 256), lambda t: (0, 0)),
        ],
        out_shape=[
            jax.ShapeDtypeStruct((8, 256), jnp.float32),
            jax.ShapeDtypeStruct((256, 256), jnp.float32),
        ],
    )(g4, u_flat, ab)


def _pass_c_body(g_ref, u_ref, ab_ref, w2_ref, f_ref, out_ref):
    g = g_ref[...]
    u = u_ref[...]
    r = _GT // 4
    u4 = jnp.concatenate([u, u, u, u], axis=1)
    u4 = jnp.broadcast_to(u4[:, None, :], (r, 4, 256)).reshape(_GT, 256)
    a1 = ab_ref[0:1, :]
    c1 = ab_ref[1:2, :]
    c2 = ab_ref[2:3, :]
    y = jnp.maximum(a1 * (g + u4) + c1, 0.0)
    z = jnp.dot(y, w2_ref[...],
                preferred_element_type=jnp.float32) + c2
    w = jnp.maximum(z, 0.0)
    t4 = jnp.sum(w.reshape(r, 4, 256), axis=1)
    s = t4[:, 0:64] + t4[:, 64:128] + t4[:, 128:192] + t4[:, 192:256]
    out_ref[0] = jnp.maximum(f_ref[0] + s.T * (1.0 / 16.0), 0.0)


def _pass_c(g4, u_flat, ab, w2blk, features):
    m4 = g4.shape[0]
    bb, c_, n_ = features.shape
    nt = n_ // (_GT // 4)
    grid = (m4 // _GT,)
    return pl.pallas_call(
        _pass_c_body,
        grid=grid,
        in_specs=[
            pl.BlockSpec((_GT, 256), lambda t: (t, 0)),
            pl.BlockSpec((_GT // 4, 64), lambda t: (t, 0)),
            pl.BlockSpec((4, 256), lambda t: (0, 0)),
            pl.BlockSpec((256, 256), lambda t: (0, 0)),
            pl.BlockSpec((1, c_, _GT // 4), lambda t: (t // nt, 0, t % nt)),
        ],
        out_specs=pl.BlockSpec((1, c_, _GT // 4), lambda t: (t // nt, 0, t % nt)),
        out_shape=jax.ShapeDtypeStruct((bb, c_, n_), jnp.float32),
    )(g4, u_flat, ab, w2blk, features)


def kernel(points, features, W1, W2, g1, b1, g2, b2):
    eps = jnp.float32(1e-5)
    bb, _, n_ = points.shape
    c_ = features.shape[1]
    k_ = 16
    m_ = bb * n_ * k_

    w1a, w1b = W1[:, :c_], W1[:, c_:]
    a_mat = w1a - w1b

    idx, u_bnc, v_bnc = _knn_uv(points, features, a_mat, w1b, k_)
    u_flat = u_bnc.reshape(bb * n_, c_)
    v_flat = v_bnc.reshape(bb * n_, c_)
    idx_flat = idx.reshape(m_)

    g_rows = _sc_gather(v_flat, idx_flat)
    g4 = g_rows.reshape(m_ // 4, 4 * c_)

    s1r, s2r = _pass_a(g4, u_flat)
    sum1 = s1r.sum(axis=0).reshape(4, c_).sum(axis=0)
    sumsq1 = s2r.sum(axis=0).reshape(4, c_).sum(axis=0)
    mf = jnp.float32(m_)
    m1 = sum1 / mf
    var1 = sumsq1 / mf - m1 * m1
    a1 = g1 / jnp.sqrt(var1 + eps)
    c1 = b1 - a1 * m1
    ab1 = jnp.stack([jnp.tile(a1, 4), jnp.tile(c1, 4)])

    syr, syy = _pass_b(g4, u_flat, ab1)
    sy = syr.sum(axis=0).reshape(4, c_).sum(axis=0)
    syy64 = (syy[0:64, 0:64] + syy[64:128, 64:128]
             + syy[128:192, 128:192] + syy[192:256, 192:256])
    m2 = (W2 @ sy) / mf
    e2 = jnp.einsum('oc,cd,od->o', W2, syy64, W2) / mf
    var2 = e2 - m2 * m2
    a2 = g2 / jnp.sqrt(var2 + eps)
    c2 = b2 - a2 * m2
    w2p = a2[:, None] * W2
    zero = jnp.zeros((c_, c_), jnp.float32)
    w2t = w2p.T
    w2blk = jnp.block([
        [w2t, zero, zero, zero],
        [zero, w2t, zero, zero],
        [zero, zero, w2t, zero],
        [zero, zero, zero, w2t],
    ])
    ab2 = jnp.concatenate(
        [ab1, jnp.tile(c2, 4)[None, :], jnp.zeros((1, 4 * c_), jnp.float32)])

    return _pass_c(g4, u_flat, ab2, w2blk, features)


# speedup vs baseline: 8.3742x; 1.0070x over previous
import functools

import jax
import jax.numpy as jnp
from jax import lax
from jax.experimental import pallas as pl
from jax.experimental.pallas import tpu as pltpu
from jax.experimental.pallas import tpu_sc as plsc

_RT = 256
_GT = 1024
_NEG = -3.0e38


def _knn_uv_body(k_, n_, pts_full, pts_rows, f_blk, a_mat, w1b, idx_ref,
                 u_ref, v_ref):
    b = pl.program_id(0)
    pf = pts_full[0]
    pr = pts_rows[0]
    xx_c = jnp.sum(pf * pf, axis=0, keepdims=True)
    xx_r = jnp.sum(pr * pr, axis=0)[:, None]
    acc = lax.dot_general(pr, pf, (((0,), (0,)), ((), ())),
                          preferred_element_type=jnp.float32)
    inner = -2.0 * acc
    P = -xx_r - inner - xx_c
    iota_f = lax.broadcasted_iota(jnp.int32, (_RT, n_), 1).astype(jnp.float32)
    cols = []
    m = jnp.max(P, axis=1, keepdims=True)
    for j in range(k_ + 1):
        jmin = jnp.min(jnp.where(P == m, iota_f, 3.0e38), axis=1,
                       keepdims=True)
        if j > 0:
            cols.append(jmin)
        P = jnp.where(iota_f == jmin, _NEG, P)
        if j < k_:
            m = jnp.max(P, axis=1, keepdims=True)
    idxf = jnp.concatenate(cols, axis=1)
    idx_ref[0] = idxf.astype(jnp.int32) + b * n_
    u_ref[0] = jnp.dot(a_mat[...], f_blk[0],
                       preferred_element_type=jnp.float32).T
    v_ref[0] = jnp.dot(w1b[...], f_blk[0],
                       preferred_element_type=jnp.float32).T


def _knn_uv(points, features, a_mat, w1b, k_):
    bb, _, n_ = points.shape
    c_ = w1b.shape[0]
    grid = (bb, n_ // _RT)
    return pl.pallas_call(
        functools.partial(_knn_uv_body, k_, n_),
        grid=grid,
        in_specs=[
            pl.BlockSpec((1, 3, n_), lambda b, t: (b, 0, 0)),
            pl.BlockSpec((1, 3, _RT), lambda b, t: (b, 0, t)),
            pl.BlockSpec((1, c_, _RT), lambda b, t: (b, 0, t)),
            pl.BlockSpec((c_, c_), lambda b, t: (0, 0)),
            pl.BlockSpec((c_, c_), lambda b, t: (0, 0)),
        ],
        out_specs=[
            pl.BlockSpec((1, _RT, k_), lambda b, t: (b, t, 0)),
            pl.BlockSpec((1, _RT, c_), lambda b, t: (b, t, 0)),
            pl.BlockSpec((1, _RT, c_), lambda b, t: (b, t, 0)),
        ],
        out_shape=[
            jax.ShapeDtypeStruct((bb, n_, k_), jnp.int32),
            jax.ShapeDtypeStruct((bb, n_, c_), jnp.float32),
            jax.ShapeDtypeStruct((bb, n_, c_), jnp.float32),
        ],
    )(points, points, features, a_mat, w1b)


def _sc_gather(v_flat, idx_flat):
    m_, c_ = idx_flat.shape[0], v_flat.shape[1]
    dt = v_flat.dtype
    info = plsc.get_sparse_core_info()
    nw = info.num_cores * info.num_subcores
    per_w = m_ // nw
    ch = 512
    n_ch = per_w // ch
    idx3 = idx_flat.reshape(nw, n_ch, ch)
    mesh = plsc.VectorSubcoreMesh(core_axis_name="c", subcore_axis_name="s")

    @functools.partial(
        pl.kernel, mesh=mesh,
        compiler_params=pltpu.CompilerParams(use_tc_tiling_on_sc=False),
        out_type=jax.ShapeDtypeStruct((m_, c_), dt),
        scratch_types=[
            pltpu.VMEM((n_ch, ch), jnp.int32),
            pltpu.VMEM((ch, c_), dt),
            pltpu.VMEM((ch, c_), dt),
            pltpu.SemaphoreType.DMA,
            pltpu.SemaphoreType.DMA,
        ],
    )
    def k(v_hbm, idx_hbm, out_hbm, idx_v, rows0, rows1, sem0, sem1):
        wid = lax.axis_index("s") * info.num_cores + lax.axis_index("c")
        base = wid * per_w
        pltpu.sync_copy(idx_hbm.at[wid], idx_v)
        bufs = (rows0, rows1)
        sems = (sem0, sem1)
        handles = [None, None]
        handles[0] = pltpu.async_copy(v_hbm.at[idx_v.at[0]], rows0, sem0)
        for i in range(n_ch):
            if i + 1 < n_ch:
                handles[(i + 1) % 2] = pltpu.async_copy(
                    v_hbm.at[idx_v.at[i + 1]], bufs[(i + 1) % 2],
                    sems[(i + 1) % 2])
            handles[i % 2].wait()
            pltpu.sync_copy(bufs[i % 2], out_hbm.at[pl.ds(base + i * ch, ch)])

    return k(v_flat, idx3)


def _pass_a_body(g_ref, u_ref, s1_ref, s2_ref):
    t = pl.program_id(0)
    g = g_ref[...].astype(jnp.float32)
    u = u_ref[...]
    r = _GT // 4
    u4 = jnp.concatenate([u, u, u, u], axis=1)
    u4 = jnp.broadcast_to(u4[:, None, :], (r, 4, 256)).reshape(_GT, 256)
    x = g + u4
    p1 = jnp.sum(x.reshape(8, _GT // 8, 256), axis=1)
    p2 = jnp.sum((x * x).reshape(8, _GT // 8, 256), axis=1)

    @pl.when(t == 0)
    def _():
        s1_ref[...] = jnp.zeros_like(s1_ref)
        s2_ref[...] = jnp.zeros_like(s2_ref)

    s1_ref[...] += p1
    s2_ref[...] += p2


def _pass_a(g4, u_flat):
    m4 = g4.shape[0]
    grid = (m4 // _GT,)
    return pl.pallas_call(
        _pass_a_body,
        grid=grid,
        in_specs=[
            pl.BlockSpec((_GT, 256), lambda t: (t, 0)),
            pl.BlockSpec((_GT // 4, 64), lambda t: (t, 0)),
        ],
        out_specs=[
            pl.BlockSpec((8, 256), lambda t: (0, 0)),
            pl.BlockSpec((8, 256), lambda t: (0, 0)),
        ],
        out_shape=[
            jax.ShapeDtypeStruct((8, 256), jnp.float32),
            jax.ShapeDtypeStruct((8, 256), jnp.float32),
        ],
    )(g4, u_flat)


def _pass_b_body(g_ref, u_ref, ab_ref, sy_ref, syy_ref):
    t = pl.program_id(0)
    g = g_ref[...].astype(jnp.float32)
    u = u_ref[...]
    r = _GT // 4
    u4 = jnp.concatenate([u, u, u, u], axis=1)
    u4 = jnp.broadcast_to(u4[:, None, :], (r, 4, 256)).reshape(_GT, 256)
    a1 = ab_ref[0:1, :]
    c1 = ab_ref[1:2, :]
    y = jnp.maximum(a1 * (g + u4) + c1, 0.0)
    py = jnp.sum(y.reshape(8, _GT // 8, 256), axis=1)
    pyy = lax.dot_general(y, y, (((0,), (0,)), ((), ())),
                          preferred_element_type=jnp.float32)

    @pl.when(t == 0)
    def _():
        sy_ref[...] = jnp.zeros_like(sy_ref)
        syy_ref[...] = jnp.zeros_like(syy_ref)

    sy_ref[...] += py
    syy_ref[...] += pyy


def _pass_b(g4, u_flat, ab):
    m4 = g4.shape[0]
    grid = (m4 // _GT,)
    return pl.pallas_call(
        _pass_b_body,
        grid=grid,
        in_specs=[
            pl.BlockSpec((_GT, 256), lambda t: (t, 0)),
            pl.BlockSpec((_GT // 4, 64), lambda t: (t, 0)),
            pl.BlockSpec((2, 256), lambda t: (0, 0)),
        ],
        out_specs=[
            pl.BlockSpec((8, 256), lambda t: (0, 0)),
            pl.BlockSpec((256, 256), lambda t: (0, 0)),
        ],
        out_shape=[
            jax.ShapeDtypeStruct((8, 256), jnp.float32),
            jax.ShapeDtypeStruct((256, 256), jnp.float32),
        ],
    )(g4, u_flat, ab)


def _pass_c_body(g_ref, u_ref, ab_ref, w2_ref, f_ref, out_ref):
    g = g_ref[...].astype(jnp.float32)
    u = u_ref[...]
    r = _GT // 4
    u4 = jnp.concatenate([u, u, u, u], axis=1)
    u4 = jnp.broadcast_to(u4[:, None, :], (r, 4, 256)).reshape(_GT, 256)
    a1 = ab_ref[0:1, :]
    c1 = ab_ref[1:2, :]
    c2 = ab_ref[2:3, :]
    y = jnp.maximum(a1 * (g + u4) + c1, 0.0)
    z = jnp.dot(y, w2_ref[...],
                preferred_element_type=jnp.float32) + c2
    w = jnp.maximum(z, 0.0)
    t4 = jnp.sum(w.reshape(r, 4, 256), axis=1)
    s = t4[:, 0:64] + t4[:, 64:128] + t4[:, 128:192] + t4[:, 192:256]
    out_ref[0] = jnp.maximum(f_ref[0] + s.T * (1.0 / 16.0), 0.0)


def _pass_c(g4, u_flat, ab, w2blk, features):
    m4 = g4.shape[0]
    bb, c_, n_ = features.shape
    nt = n_ // (_GT // 4)
    grid = (m4 // _GT,)
    return pl.pallas_call(
        _pass_c_body,
        grid=grid,
        in_specs=[
            pl.BlockSpec((_GT, 256), lambda t: (t, 0)),
            pl.BlockSpec((_GT // 4, 64), lambda t: (t, 0)),
            pl.BlockSpec((4, 256), lambda t: (0, 0)),
            pl.BlockSpec((256, 256), lambda t: (0, 0)),
            pl.BlockSpec((1, c_, _GT // 4), lambda t: (t // nt, 0, t % nt)),
        ],
        out_specs=pl.BlockSpec((1, c_, _GT // 4), lambda t: (t // nt, 0, t % nt)),
        out_shape=jax.ShapeDtypeStruct((bb, c_, n_), jnp.float32),
    )(g4, u_flat, ab, w2blk, features)


def kernel(points, features, W1, W2, g1, b1, g2, b2):
    eps = jnp.float32(1e-5)
    bb, _, n_ = points.shape
    c_ = features.shape[1]
    k_ = 16
    m_ = bb * n_ * k_

    w1a, w1b = W1[:, :c_], W1[:, c_:]
    a_mat = w1a - w1b

    idx, u_bnc, v_bnc = _knn_uv(points, features, a_mat, w1b, k_)
    u_flat = u_bnc.reshape(bb * n_, c_)
    v_flat = v_bnc.reshape(bb * n_, c_)
    idx_flat = idx.reshape(m_)

    g_rows = _sc_gather(v_flat.astype(jnp.bfloat16), idx_flat)
    g4 = g_rows.reshape(m_ // 4, 4 * c_)

    s1r, s2r = _pass_a(g4, u_flat)
    sum1 = s1r.sum(axis=0).reshape(4, c_).sum(axis=0)
    sumsq1 = s2r.sum(axis=0).reshape(4, c_).sum(axis=0)
    mf = jnp.float32(m_)
    m1 = sum1 / mf
    var1 = sumsq1 / mf - m1 * m1
    a1 = g1 / jnp.sqrt(var1 + eps)
    c1 = b1 - a1 * m1
    ab1 = jnp.stack([jnp.tile(a1, 4), jnp.tile(c1, 4)])

    syr, syy = _pass_b(g4, u_flat, ab1)
    sy = syr.sum(axis=0).reshape(4, c_).sum(axis=0)
    syy64 = (syy[0:64, 0:64] + syy[64:128, 64:128]
             + syy[128:192, 128:192] + syy[192:256, 192:256])
    m2 = (W2 @ sy) / mf
    e2 = jnp.einsum('oc,cd,od->o', W2, syy64, W2) / mf
    var2 = e2 - m2 * m2
    a2 = g2 / jnp.sqrt(var2 + eps)
    c2 = b2 - a2 * m2
    w2p = a2[:, None] * W2
    zero = jnp.zeros((c_, c_), jnp.float32)
    w2t = w2p.T
    w2blk = jnp.block([
        [w2t, zero, zero, zero],
        [zero, w2t, zero, zero],
        [zero, zero, w2t, zero],
        [zero, zero, zero, w2t],
    ])
    ab2 = jnp.concatenate(
        [ab1, jnp.tile(c2, 4)[None, :], jnp.zeros((1, 4 * c_), jnp.float32)])

    return _pass_c(g4, u_flat, ab2, w2blk, features)
